# Initial kernel scaffold; baseline (speedup 1.0000x reference)
import functools

import jax
import jax.numpy as jnp
from jax import lax
from jax.experimental import pallas as pl
from jax.experimental.pallas import tpu as pltpu
from jax.experimental.pallas import tpu_sc as plsc

N = 10000
E = 320000
D = 128

NC = 2
NS = 16
EPC = E // NC
EPS = EPC // NS
CH = 400
NCHUNK = EPS // CH
RPZ = N // NS

BN = 2048
GRID = (N + BN - 1) // BN


def _dot_t(a, w):
    return lax.dot_general(a, w, (((1,), (1,)), ((), ())),
                           preferred_element_type=jnp.float32)


def _pre_body(x_ref, wl_ref, wr_ref, b_ref, y_ref, r_ref):
    xb = x_ref[...]
    y_ref[...] = _dot_t(xb, wl_ref[...])
    r_ref[...] = _dot_t(xb, wr_ref[...]) + b_ref[...]


def _tc_pre(x, wl, wr, b):
    return pl.pallas_call(
        _pre_body,
        grid=(GRID,),
        in_specs=[
            pl.BlockSpec((BN, D), lambda i: (i, 0)),
            pl.BlockSpec((D, D), lambda i: (0, 0)),
            pl.BlockSpec((D, D), lambda i: (0, 0)),
            pl.BlockSpec((1, D), lambda i: (0, 0)),
        ],
        out_specs=[
            pl.BlockSpec((BN, D), lambda i: (i, 0)),
            pl.BlockSpec((BN, D), lambda i: (i, 0)),
        ],
        out_shape=[
            jax.ShapeDtypeStruct((N, D), jnp.float32),
            jax.ShapeDtypeStruct((N, D), jnp.float32),
        ],
    )(x, wl, wr, b)


def _combine_pre_body(s_ref, cnt_ref, r_ref, wl_ref, wr_ref, b_ref,
                      y_ref, r2_ref):
    summed = s_ref[0] + s_ref[1]
    cnt = cnt_ref[0, 0, :] + cnt_ref[1, 0, :]
    inv = 1.0 / jnp.maximum(cnt, 1.0)
    h = jnp.maximum(summed * inv[:, None] + r_ref[...], 0.0)
    y_ref[...] = _dot_t(h, wl_ref[...])
    r2_ref[...] = _dot_t(h, wr_ref[...]) + b_ref[...]


def _tc_combine_pre(s, cnt, r, wl, wr, b):
    return pl.pallas_call(
        _combine_pre_body,
        grid=(GRID,),
        in_specs=[
            pl.BlockSpec((NC, BN, D), lambda i: (0, i, 0)),
            pl.BlockSpec((NC, 8, BN), lambda i: (0, 0, i)),
            pl.BlockSpec((BN, D), lambda i: (i, 0)),
            pl.BlockSpec((D, D), lambda i: (0, 0)),
            pl.BlockSpec((D, D), lambda i: (0, 0)),
            pl.BlockSpec((1, D), lambda i: (0, 0)),
        ],
        out_specs=[
            pl.BlockSpec((BN, D), lambda i: (i, 0)),
            pl.BlockSpec((BN, D), lambda i: (i, 0)),
        ],
        out_shape=[
            jax.ShapeDtypeStruct((N, D), jnp.float32),
            jax.ShapeDtypeStruct((N, D), jnp.float32),
        ],
    )(s, cnt, r, wl, wr, b)


def _final_body(s_ref, cnt_ref, r_ref, o_ref):
    summed = s_ref[0] + s_ref[1]
    cnt = cnt_ref[0, 0, :] + cnt_ref[1, 0, :]
    inv = 1.0 / jnp.maximum(cnt, 1.0)
    o_ref[...] = summed * inv[:, None] + r_ref[...]


def _tc_final(s, cnt, r):
    return pl.pallas_call(
        _final_body,
        grid=(GRID,),
        in_specs=[
            pl.BlockSpec((NC, BN, D), lambda i: (0, i, 0)),
            pl.BlockSpec((NC, 8, BN), lambda i: (0, 0, i)),
            pl.BlockSpec((BN, D), lambda i: (i, 0)),
        ],
        out_specs=pl.BlockSpec((BN, D), lambda i: (i, 0)),
        out_shape=jax.ShapeDtypeStruct((N, D), jnp.float32),
    )(s, cnt, r)


def _make_segsum(with_counts):
    mesh = plsc.VectorSubcoreMesh(core_axis_name="c", subcore_axis_name="s")
    out_type = [jax.ShapeDtypeStruct((NC, N, D), jnp.float32)]
    if with_counts:
        out_type.append(jax.ShapeDtypeStruct((NC, 8, N), jnp.float32))
    scratch = [
        pltpu.VMEM((CH,), jnp.int32),
        pltpu.VMEM((CH,), jnp.int32),
        pltpu.VMEM((CH, D), jnp.float32),
        pltpu.VMEM((CH,), jnp.float32),
        pltpu.VMEM_SHARED((N, D), jnp.float32),
        pltpu.VMEM_SHARED((N,), jnp.float32),
        pltpu.SemaphoreType.DMA,
    ]

    def body(y_hbm, src_hbm, dst_hbm, zf_hbm, zc_hbm, *rest):
        if with_counts:
            (s_out, cnt_out, srcb, dstb, rows, ones, acc, cacc, sem) = rest
        else:
            (s_out, srcb, dstb, rows, ones, acc, cacc, sem) = rest
            cnt_out = None
        c = lax.axis_index("c")
        s = lax.axis_index("s")
        pltpu.sync_copy(zf_hbm.at[pl.ds(s * RPZ, RPZ)],
                        acc.at[pl.ds(s * RPZ, RPZ)])
        if with_counts:
            @pl.when(s == 0)
            def _():
                pltpu.sync_copy(zc_hbm, cacc)

            def fill(i, carry):
                ones[pl.ds(i * 16, 16)] = jnp.ones((16,), jnp.float32)
                return carry
            lax.fori_loop(0, CH // 16, fill, 0)
        plsc.subcore_barrier()

        base = (c * NS + s) * EPS

        def chunk(t, carry):
            off = base + t * CH
            pltpu.sync_copy(src_hbm.at[pl.ds(off, CH)], srcb)
            pltpu.sync_copy(dst_hbm.at[pl.ds(off, CH)], dstb)
            pltpu.async_copy(y_hbm.at[srcb], rows, sem).wait()
            pltpu.sync_copy(rows, acc.at[dstb], add=True)
            if with_counts:
                pltpu.sync_copy(ones, cacc.at[dstb], add=True)
            return carry
        lax.fori_loop(0, NCHUNK, chunk, 0)
        plsc.subcore_barrier()

        pltpu.sync_copy(acc.at[pl.ds(s * RPZ, RPZ)],
                        s_out.at[c, pl.ds(s * RPZ, RPZ)])
        if with_counts:
            @pl.when(s == 0)
            def _():
                pltpu.sync_copy(cacc, cnt_out.at[c, 0])

    return pl.kernel(body, out_type=out_type, mesh=mesh,
                     scratch_types=scratch)


_segsum_counts = _make_segsum(True)
_segsum_plain = _make_segsum(False)


def kernel(x, edge_index, W1_l, b1_l, W1_r, W2_l, b2_l, W2_r):
    src = edge_index[0]
    dst = edge_index[1]
    zf = jnp.zeros((N, D), jnp.float32)
    zc = jnp.zeros((N,), jnp.float32)

    y1, r1 = _tc_pre(x, W1_l, W1_r, b1_l.reshape(1, D))
    s1, cnt = _segsum_counts(y1, src, dst, zf, zc)
    y2, r2 = _tc_combine_pre(s1, cnt, r1, W2_l, W2_r, b2_l.reshape(1, D))
    (s2,) = _segsum_plain(y2, src, dst, zf, zc)
    return _tc_final(s2, cnt, r2)


# speedup vs baseline: 3.5239x; 3.5239x over previous
import jax
import jax.numpy as jnp
from jax import lax
from jax.experimental import pallas as pl
from jax.experimental.pallas import tpu as pltpu
from jax.experimental.pallas import tpu_sc as plsc

N = 10000
E = 320000
D = 128
DH = D // 2

NC = 2
NS = 16
EPS = E // NS
CH = 80
NCHUNK = EPS // CH
NP = 10240
RPZ = NP // NS

BN = 2048
GRID = NP // BN


def _dot_t(a, w):
    return lax.dot_general(a, w, (((1,), (1,)), ((), ())),
                           preferred_element_type=jnp.float32)


def _pre_body(x_ref, wl_ref, wr_ref, b_ref, ya_ref, yb_ref, r_ref):
    xb = x_ref[...]
    y = _dot_t(xb, wl_ref[...])
    ya_ref[...] = y[:, :DH]
    yb_ref[...] = y[:, DH:]
    r_ref[...] = _dot_t(xb, wr_ref[...]) + b_ref[...]


def _tc_pre(x, wl, wr, b):
    return pl.pallas_call(
        _pre_body,
        grid=(GRID,),
        in_specs=[
            pl.BlockSpec((BN, D), lambda i: (i, 0)),
            pl.BlockSpec((D, D), lambda i: (0, 0)),
            pl.BlockSpec((D, D), lambda i: (0, 0)),
            pl.BlockSpec((1, D), lambda i: (0, 0)),
        ],
        out_specs=[
            pl.BlockSpec((BN, DH), lambda i: (i, 0)),
            pl.BlockSpec((BN, DH), lambda i: (i, 0)),
            pl.BlockSpec((BN, D), lambda i: (i, 0)),
        ],
        out_shape=[
            jax.ShapeDtypeStruct((N, DH), jnp.float32),
            jax.ShapeDtypeStruct((N, DH), jnp.float32),
            jax.ShapeDtypeStruct((N, D), jnp.float32),
        ],
    )(x, wl, wr, b)


def _combine(s_ref, cnt_ref, r_ref):
    summed = jnp.concatenate([s_ref[0], s_ref[1]], axis=1)
    inv = 1.0 / jnp.maximum(cnt_ref[0, :], 1.0)
    return summed * inv[:, None] + r_ref[...]


def _combine_pre_body(s_ref, cnt_ref, r_ref, wl_ref, wr_ref, b_ref,
                      ya_ref, yb_ref, r2_ref):
    h = jnp.maximum(_combine(s_ref, cnt_ref, r_ref), 0.0)
    y = _dot_t(h, wl_ref[...])
    ya_ref[...] = y[:, :DH]
    yb_ref[...] = y[:, DH:]
    r2_ref[...] = _dot_t(h, wr_ref[...]) + b_ref[...]


def _tc_combine_pre(s, cnt, r, wl, wr, b):
    return pl.pallas_call(
        _combine_pre_body,
        grid=(GRID,),
        in_specs=[
            pl.BlockSpec((NC, BN, DH), lambda i: (0, i, 0)),
            pl.BlockSpec((8, BN), lambda i: (0, i)),
            pl.BlockSpec((BN, D), lambda i: (i, 0)),
            pl.BlockSpec((D, D), lambda i: (0, 0)),
            pl.BlockSpec((D, D), lambda i: (0, 0)),
            pl.BlockSpec((1, D), lambda i: (0, 0)),
        ],
        out_specs=[
            pl.BlockSpec((BN, DH), lambda i: (i, 0)),
            pl.BlockSpec((BN, DH), lambda i: (i, 0)),
            pl.BlockSpec((BN, D), lambda i: (i, 0)),
        ],
        out_shape=[
            jax.ShapeDtypeStruct((N, DH), jnp.float32),
            jax.ShapeDtypeStruct((N, DH), jnp.float32),
            jax.ShapeDtypeStruct((N, D), jnp.float32),
        ],
    )(s, cnt, r, wl, wr, b)


def _final_body(s_ref, cnt_ref, r_ref, o_ref):
    o_ref[...] = _combine(s_ref, cnt_ref, r_ref)


def _tc_final(s, cnt, r):
    return pl.pallas_call(
        _final_body,
        grid=(GRID,),
        in_specs=[
            pl.BlockSpec((NC, BN, DH), lambda i: (0, i, 0)),
            pl.BlockSpec((8, BN), lambda i: (0, i)),
            pl.BlockSpec((BN, D), lambda i: (i, 0)),
        ],
        out_specs=pl.BlockSpec((BN, D), lambda i: (i, 0)),
        out_shape=jax.ShapeDtypeStruct((N, D), jnp.float32),
    )(s, cnt, r)


def _make_segsum(with_counts):
    mesh = plsc.VectorSubcoreMesh(core_axis_name="c", subcore_axis_name="s",
                                  num_cores=NC, num_subcores=NS)
    out_type = [jax.ShapeDtypeStruct((NC, NP, DH), jnp.float32)]
    if with_counts:
        out_type.append(jax.ShapeDtypeStruct((8, NP), jnp.float32))
    scratch = [
        pltpu.VMEM((CH,), jnp.int32),
        pltpu.VMEM((CH,), jnp.int32),
        pltpu.VMEM((CH, DH), jnp.float32),
        pltpu.VMEM((CH,), jnp.float32),
        pltpu.VMEM_SHARED((NP, DH), jnp.float32),
        pltpu.VMEM_SHARED((NP,), jnp.float32),
        pltpu.SemaphoreType.DMA,
    ]

    def body(ya_hbm, yb_hbm, src_hbm, dst_hbm, zf_hbm, zc_hbm, *rest):
        if with_counts:
            (s_out, cnt_out, srcb, dstb, rows, ones, acc, cacc, sem) = rest
        else:
            (s_out, srcb, dstb, rows, ones, acc, cacc, sem) = rest
            cnt_out = None
        c = lax.axis_index("c")
        s = lax.axis_index("s")
        pltpu.sync_copy(zf_hbm.at[pl.ds(s * RPZ, RPZ)],
                        acc.at[pl.ds(s * RPZ, RPZ)])
        if with_counts:
            @pl.when(jnp.logical_and(c == 0, s == 0))
            def _():
                pltpu.sync_copy(zc_hbm, cacc)

            def fill(i, carry):
                ones[pl.ds(i * 16, 16)] = jnp.ones((16,), jnp.float32)
                return carry
            lax.fori_loop(0, CH // 16, fill, 0)
        plsc.subcore_barrier()

        base = s * EPS

        def chunk(t, carry):
            off = base + t * CH
            pltpu.sync_copy(src_hbm.at[pl.ds(off, CH)], srcb)
            pltpu.sync_copy(dst_hbm.at[pl.ds(off, CH)], dstb)

            @pl.when(c == 0)
            def _():
                pltpu.async_copy(ya_hbm.at[srcb], rows, sem).wait()

            @pl.when(c == 1)
            def _():
                pltpu.async_copy(yb_hbm.at[srcb], rows, sem).wait()

            pltpu.sync_copy(rows, acc.at[dstb], add=True)
            if with_counts:
                @pl.when(c == 0)
                def _():
                    pltpu.sync_copy(ones, cacc.at[dstb], add=True)
            return carry
        lax.fori_loop(0, NCHUNK, chunk, 0)
        plsc.subcore_barrier()

        pltpu.sync_copy(acc.at[pl.ds(s * RPZ, RPZ)],
                        s_out.at[c, pl.ds(s * RPZ, RPZ)])
        if with_counts:
            @pl.when(jnp.logical_and(c == 0, s == 0))
            def _():
                pltpu.sync_copy(cacc, cnt_out.at[0])

    return pl.kernel(body, out_type=out_type, mesh=mesh,
                     scratch_types=scratch,
                     compiler_params=pltpu.CompilerParams(
                         use_tc_tiling_on_sc=False))


_segsum_counts = _make_segsum(True)
_segsum_plain = _make_segsum(False)


def kernel(x, edge_index, W1_l, b1_l, W1_r, W2_l, b2_l, W2_r):
    src = edge_index[0]
    dst = edge_index[1]
    zf = jnp.zeros((NP, DH), jnp.float32)
    zc = jnp.zeros((NP,), jnp.float32)

    y1a, y1b, r1 = _tc_pre(x, W1_l, W1_r, b1_l.reshape(1, D))
    s1, cnt = _segsum_counts(y1a, y1b, src, dst, zf, zc)
    y2a, y2b, r2 = _tc_combine_pre(s1, cnt, r1, W2_l, W2_r, b2_l.reshape(1, D))
    (s2,) = _segsum_plain(y2a, y2b, src, dst, zf, zc)
    return _tc_final(s2, cnt, r2)


# speedup vs baseline: 9.7972x; 2.7802x over previous
import jax
import jax.numpy as jnp
from jax import lax
from jax.experimental import pallas as pl
from jax.experimental.pallas import tpu as pltpu
from jax.experimental.pallas import tpu_sc as plsc

N = 10000
E = 320000
D = 128
DH = D // 2

NC = 2
NS = 16
EPS = E // NS
CH = 80
NCHUNK = EPS // CH
NP = 10240
RPZ = NP // NS

BN = 2048
GRID = NP // BN


def _dot_t(a, w):
    return lax.dot_general(a, w, (((1,), (1,)), ((), ())),
                           preferred_element_type=jnp.float32)


def _pre_body(x_ref, wl_ref, wr_ref, b_ref, ya_ref, yb_ref, r_ref):
    xb = x_ref[...]
    y = _dot_t(xb, wl_ref[...])
    ya_ref[...] = y[:, :DH]
    yb_ref[...] = y[:, DH:]
    r_ref[...] = _dot_t(xb, wr_ref[...]) + b_ref[...]


def _tc_pre(x, wl, wr, b):
    return pl.pallas_call(
        _pre_body,
        grid=(GRID,),
        in_specs=[
            pl.BlockSpec((BN, D), lambda i: (i, 0)),
            pl.BlockSpec((D, D), lambda i: (0, 0)),
            pl.BlockSpec((D, D), lambda i: (0, 0)),
            pl.BlockSpec((1, D), lambda i: (0, 0)),
        ],
        out_specs=[
            pl.BlockSpec((BN, DH), lambda i: (i, 0)),
            pl.BlockSpec((BN, DH), lambda i: (i, 0)),
            pl.BlockSpec((BN, D), lambda i: (i, 0)),
        ],
        out_shape=[
            jax.ShapeDtypeStruct((N, DH), jnp.float32),
            jax.ShapeDtypeStruct((N, DH), jnp.float32),
            jax.ShapeDtypeStruct((N, D), jnp.float32),
        ],
    )(x, wl, wr, b)


def _combine(s_ref, cnt_ref, r_ref):
    summed = jnp.concatenate([s_ref[0], s_ref[1]], axis=1)
    inv = 1.0 / jnp.maximum(cnt_ref[0, :], 1.0)
    return summed * inv[:, None] + r_ref[...]


def _combine_pre_body(s_ref, cnt_ref, r_ref, wl_ref, wr_ref, b_ref,
                      ya_ref, yb_ref, r2_ref):
    h = jnp.maximum(_combine(s_ref, cnt_ref, r_ref), 0.0)
    y = _dot_t(h, wl_ref[...])
    ya_ref[...] = y[:, :DH]
    yb_ref[...] = y[:, DH:]
    r2_ref[...] = _dot_t(h, wr_ref[...]) + b_ref[...]


def _tc_combine_pre(s, cnt, r, wl, wr, b):
    return pl.pallas_call(
        _combine_pre_body,
        grid=(GRID,),
        in_specs=[
            pl.BlockSpec((NC, BN, DH), lambda i: (0, i, 0)),
            pl.BlockSpec((8, BN), lambda i: (0, i)),
            pl.BlockSpec((BN, D), lambda i: (i, 0)),
            pl.BlockSpec((D, D), lambda i: (0, 0)),
            pl.BlockSpec((D, D), lambda i: (0, 0)),
            pl.BlockSpec((1, D), lambda i: (0, 0)),
        ],
        out_specs=[
            pl.BlockSpec((BN, DH), lambda i: (i, 0)),
            pl.BlockSpec((BN, DH), lambda i: (i, 0)),
            pl.BlockSpec((BN, D), lambda i: (i, 0)),
        ],
        out_shape=[
            jax.ShapeDtypeStruct((N, DH), jnp.float32),
            jax.ShapeDtypeStruct((N, DH), jnp.float32),
            jax.ShapeDtypeStruct((N, D), jnp.float32),
        ],
    )(s, cnt, r, wl, wr, b)


def _final_body(s_ref, cnt_ref, r_ref, o_ref):
    o_ref[...] = _combine(s_ref, cnt_ref, r_ref)


def _tc_final(s, cnt, r):
    return pl.pallas_call(
        _final_body,
        grid=(GRID,),
        in_specs=[
            pl.BlockSpec((NC, BN, DH), lambda i: (0, i, 0)),
            pl.BlockSpec((8, BN), lambda i: (0, i)),
            pl.BlockSpec((BN, D), lambda i: (i, 0)),
        ],
        out_specs=pl.BlockSpec((BN, D), lambda i: (i, 0)),
        out_shape=jax.ShapeDtypeStruct((N, D), jnp.float32),
    )(s, cnt, r)


def _make_segsum(with_counts):
    mesh = plsc.VectorSubcoreMesh(core_axis_name="c", subcore_axis_name="s",
                                  num_cores=NC, num_subcores=NS)
    out_type = [jax.ShapeDtypeStruct((NC, NP, DH), jnp.float32)]
    if with_counts:
        out_type.append(jax.ShapeDtypeStruct((8, NP), jnp.float32))
    scratch = [
        pltpu.VMEM((EPS,), jnp.int32),
        pltpu.VMEM((NCHUNK, CH), jnp.int32),
        pltpu.VMEM((CH, DH), jnp.float32),
        pltpu.VMEM((CH, DH), jnp.float32),
        pltpu.VMEM((CH,), jnp.float32),
        pltpu.VMEM_SHARED((NP, DH), jnp.float32),
        pltpu.VMEM_SHARED((NP,), jnp.float32),
        pltpu.SemaphoreType.DMA,
        pltpu.SemaphoreType.DMA,
    ]

    def body(ya_hbm, yb_hbm, src_hbm, dst_hbm, zf_hbm, zc_hbm, *rest):
        if with_counts:
            (s_out, cnt_out, srca, dsta, rows0, rows1, ones, acc, cacc,
             sem0, sem1) = rest
        else:
            (s_out, srca, dsta, rows0, rows1, ones, acc, cacc,
             sem0, sem1) = rest
            cnt_out = None
        rows = (rows0, rows1)
        sems = (sem0, sem1)
        c = lax.axis_index("c")
        s = lax.axis_index("s")
        pltpu.sync_copy(zf_hbm.at[pl.ds(s * RPZ, RPZ)],
                        acc.at[pl.ds(s * RPZ, RPZ)])
        pltpu.sync_copy(src_hbm.at[pl.ds(s * EPS, EPS)], srca)
        pltpu.sync_copy(dst_hbm.at[s], dsta)
        if with_counts:
            @pl.when(jnp.logical_and(c == 0, s == 0))
            def _():
                pltpu.sync_copy(zc_hbm, cacc)

            def fill(i, carry):
                ones[pl.ds(i * 16, 16)] = jnp.ones((16,), jnp.float32)
                return carry
            lax.fori_loop(0, CH // 16, fill, 0)
        plsc.subcore_barrier()

        def issue_gather(t, b):
            idx = srca.at[pl.ds(t * CH, CH)]

            @pl.when(c == 0)
            def _():
                pltpu.async_copy(ya_hbm.at[idx], rows[b], sems[b])

            @pl.when(c == 1)
            def _():
                pltpu.async_copy(yb_hbm.at[idx], rows[b], sems[b])

        issue_gather(0, 0)
        issue_gather(1, 1)

        def outer(i, carry):
            for b in range(2):
                t = i * 2 + b
                pltpu.make_async_copy(ya_hbm.at[pl.ds(0, CH)], rows[b],
                                      sems[b]).wait()
                pltpu.sync_copy(rows[b], acc.at[dsta.at[t]], add=True)
                if with_counts:
                    @pl.when(c == 0)
                    def _():
                        pltpu.sync_copy(ones, cacc.at[dsta.at[t]], add=True)

                @pl.when(t + 2 < NCHUNK)
                def _():
                    issue_gather(t + 2, b)
            return carry
        lax.fori_loop(0, NCHUNK // 2, outer, 0)
        plsc.subcore_barrier()

        pltpu.sync_copy(acc.at[pl.ds(s * RPZ, RPZ)],
                        s_out.at[c, pl.ds(s * RPZ, RPZ)])
        if with_counts:
            @pl.when(jnp.logical_and(c == 0, s == 0))
            def _():
                pltpu.sync_copy(cacc, cnt_out.at[0])

    return pl.kernel(body, out_type=out_type, mesh=mesh,
                     scratch_types=scratch,
                     compiler_params=pltpu.CompilerParams(
                         use_tc_tiling_on_sc=False))


_segsum_counts = _make_segsum(True)
_segsum_plain = _make_segsum(False)


def kernel(x, edge_index, W1_l, b1_l, W1_r, W2_l, b2_l, W2_r):
    src = edge_index[0]
    dst = edge_index[1].reshape(NS, NCHUNK, CH)
    zf = jnp.zeros((NP, DH), jnp.float32)
    zc = jnp.zeros((NP,), jnp.float32)

    y1a, y1b, r1 = _tc_pre(x, W1_l, W1_r, b1_l.reshape(1, D))
    s1, cnt = _segsum_counts(y1a, y1b, src, dst, zf, zc)
    y2a, y2b, r2 = _tc_combine_pre(s1, cnt, r1, W2_l, W2_r, b2_l.reshape(1, D))
    (s2,) = _segsum_plain(y2a, y2b, src, dst, zf, zc)
    return _tc_final(s2, cnt, r2)
